# baseline (speedup 1.0000x reference)
import functools

import jax
import jax.numpy as jnp
from jax import lax
from jax.experimental import pallas as pl
from jax.experimental.pallas import tpu as pltpu
from jax.experimental.pallas import tpu_sc as plsc

B = 4096
L = 50
H = 64
N = B * L
CHUNK = 128
NCHUNK = N // CHUNK
NC = 2
NS = 16
NW = NC * NS
CPW = NCHUNK // NW
IPW = N // NW
NBUF = 2
LANES = 16


@functools.partial(
    pl.kernel,
    out_type=jax.ShapeDtypeStruct((N, 2 * H), jnp.float32),
    mesh=plsc.VectorSubcoreMesh(core_axis_name="c", subcore_axis_name="s"),
    scratch_types=[
        pltpu.VMEM((IPW,), jnp.int32),
        pltpu.VMEM((IPW,), jnp.int32),
        pltpu.VMEM((IPW,), jnp.int32),
        pltpu.VMEM((IPW,), jnp.int32),
        pltpu.VMEM((NBUF, CHUNK, 2 * H), jnp.float32),
        pltpu.VMEM((NBUF, CHUNK, 2 * H), jnp.float32),
        pltpu.VMEM((NBUF, CHUNK, 2 * H), jnp.float32),
        pltpu.SemaphoreType.DMA((NBUF,)),
        pltpu.SemaphoreType.DMA((NBUF,)),
    ],
)
def _emb_gather(form_idx_hbm, vice_idx_hbm, wform_hbm, wvice_hbm, out_hbm,
                fidx_v, vidx_v, frow_v, vrow_v, comb_f, comb_v, wbuf,
                gsem, wsem):
    wid = lax.axis_index("s") * NC + lax.axis_index("c")
    base = wid * CPW
    pltpu.sync_copy(form_idx_hbm.at[pl.ds(wid * IPW, IPW)], fidx_v)
    pltpu.sync_copy(vice_idx_hbm.at[pl.ds(wid * IPW, IPW)], vidx_v)

    def shiftbody(g, carry):
        s = pl.ds(g * LANES, LANES)
        frow_v[s] = jax.lax.shift_right_logical(fidx_v[s], 1)
        vrow_v[s] = jax.lax.shift_right_logical(vidx_v[s], 1)
        return carry

    lax.fori_loop(0, IPW // LANES, shiftbody, 0)

    def fire_gather(v, b):
        pltpu.async_copy(wform_hbm.at[frow_v.at[pl.ds(v * CHUNK, CHUNK)]],
                         comb_f.at[b], gsem.at[b])
        pltpu.async_copy(wvice_hbm.at[vrow_v.at[pl.ds(v * CHUNK, CHUNK)]],
                         comb_v.at[b], gsem.at[b])

    def wait_gather(b):
        pltpu.make_async_copy(
            wform_hbm.at[pl.ds(0, CHUNK)], comb_f.at[b], gsem.at[b]).wait()
        pltpu.make_async_copy(
            wvice_hbm.at[pl.ds(0, CHUNK)], comb_v.at[b], gsem.at[b]).wait()

    def pack(v, b):
        def grpbody(g, carry):
            pf = fidx_v[pl.ds(v * CHUNK + g * LANES, LANES)]
            pv = vidx_v[pl.ds(v * CHUNK + g * LANES, LANES)]
            for j in range(LANES):
                r = g * LANES + j
                fo = (pf[j] & 1) * H
                vo = (pv[j] & 1) * H
                for k in range(H // LANES):
                    wbuf.at[b, r][pl.ds(LANES * k, LANES)] = (
                        comb_f.at[b, r][pl.ds(fo + LANES * k, LANES)])
                    wbuf.at[b, r][pl.ds(H + LANES * k, LANES)] = (
                        comb_v.at[b, r][pl.ds(vo + LANES * k, LANES)])
            return carry

        lax.fori_loop(0, CHUNK // LANES, grpbody, 0)

    def fire_write(v, b):
        pltpu.async_copy(wbuf.at[b],
                         out_hbm.at[pl.ds((base + v) * CHUNK, CHUNK)],
                         wsem.at[b])

    def wait_write(b):
        pltpu.make_async_copy(
            wbuf.at[b], out_hbm.at[pl.ds(base * CHUNK, CHUNK)],
            wsem.at[b]).wait()

    fire_gather(0, 0)
    fire_gather(1, 1)

    def body(i, carry):
        for b in range(NBUF):
            v = NBUF * i + b
            wait_gather(b)

            @pl.when(v >= NBUF)
            def _():
                wait_write(b)

            pack(v, b)

            @pl.when(v + NBUF < CPW)
            def _():
                fire_gather(v + NBUF, b)

            fire_write(v, b)
        return carry

    lax.fori_loop(0, CPW // NBUF, body, 0)

    for b in range(NBUF):
        wait_write(b)


def kernel(form_idx, vice_idx, W_form, W_vice):
    fi = form_idx.astype(jnp.int32).T.reshape(N)
    vi = vice_idx.astype(jnp.int32).T.reshape(N)
    wf2 = W_form.reshape(W_form.shape[0] // 2, 2 * H)
    wv2 = W_vice.reshape(W_vice.shape[0] // 2, 2 * H)
    out = _emb_gather(fi, vi, wf2, wv2)
    return out.reshape(L, B, 2 * H).transpose(1, 0, 2)


# speedup vs baseline: 1.1610x; 1.1610x over previous
import functools

import jax
import jax.numpy as jnp
from jax import lax
from jax.experimental import pallas as pl
from jax.experimental.pallas import tpu as pltpu
from jax.experimental.pallas import tpu_sc as plsc

B = 4096
L = 50
H = 64
N = B * L
CHUNK = 128
NCHUNK = N // CHUNK
NC = 2
NS = 16
NW = NC * NS
CPW = NCHUNK // NW
IPW = N // NW
NBUF = 4


@functools.partial(
    pl.kernel,
    out_type=jax.ShapeDtypeStruct((N, 2 * H), jnp.float32),
    mesh=plsc.VectorSubcoreMesh(core_axis_name="c", subcore_axis_name="s"),
    compiler_params=pltpu.CompilerParams(use_tc_tiling_on_sc=False),
    scratch_types=[
        pltpu.VMEM((IPW,), jnp.int32),
        pltpu.VMEM((IPW,), jnp.int32),
        pltpu.VMEM((NBUF, CHUNK, H), jnp.float32),
        pltpu.VMEM((NBUF, CHUNK, H), jnp.float32),
        pltpu.SemaphoreType.DMA((NBUF,)),
        pltpu.SemaphoreType.DMA((NBUF,)),
    ],
)
def _emb_gather(form_idx_hbm, vice_idx_hbm, wform_hbm, wvice_hbm, out_hbm,
                fidx_v, vidx_v, frows, vrows, gsem, wsem):
    wid = lax.axis_index("s") * NC + lax.axis_index("c")
    base = wid * CPW
    pltpu.sync_copy(form_idx_hbm.at[pl.ds(wid * IPW, IPW)], fidx_v)
    pltpu.sync_copy(vice_idx_hbm.at[pl.ds(wid * IPW, IPW)], vidx_v)

    def fire_gather(v, b):
        pltpu.async_copy(wform_hbm.at[fidx_v.at[pl.ds(v * CHUNK, CHUNK)]],
                         frows.at[b], gsem.at[b])
        pltpu.async_copy(wvice_hbm.at[vidx_v.at[pl.ds(v * CHUNK, CHUNK)]],
                         vrows.at[b], gsem.at[b])

    def wait_gather(b):
        pltpu.make_async_copy(
            wform_hbm.at[pl.ds(0, CHUNK)], frows.at[b], gsem.at[b]).wait()
        pltpu.make_async_copy(
            wvice_hbm.at[pl.ds(0, CHUNK)], vrows.at[b], gsem.at[b]).wait()

    def fire_write(v, b):
        row0 = (base + v) * CHUNK
        pltpu.async_copy(frows.at[b],
                         out_hbm.at[pl.ds(row0, CHUNK), pl.ds(0, H)],
                         wsem.at[b])
        pltpu.async_copy(vrows.at[b],
                         out_hbm.at[pl.ds(row0, CHUNK), pl.ds(H, H)],
                         wsem.at[b])

    def wait_write(b):
        row0 = base * CHUNK
        pltpu.make_async_copy(
            frows.at[b], out_hbm.at[pl.ds(row0, CHUNK), pl.ds(0, H)],
            wsem.at[b]).wait()
        pltpu.make_async_copy(
            vrows.at[b], out_hbm.at[pl.ds(row0, CHUNK), pl.ds(H, H)],
            wsem.at[b]).wait()

    fire_gather(0, 0)
    fire_gather(1, 1)

    def body(i, carry):
        for b in range(NBUF):
            v = NBUF * i + b
            nb = (b + 2) % NBUF
            wait_gather(b)
            fire_write(v, b)

            @pl.when(v >= 2)
            def _():
                wait_write(nb)

            fire_gather(v + 2, nb)
        return carry

    lax.fori_loop(0, CPW // NBUF, body, 0)

    for v, b in ((CPW - 2, 0), (CPW - 1, 1)):
        wait_gather(b)
        fire_write(v, b)

    for b in range(NBUF):
        wait_write(b)


def kernel(form_idx, vice_idx, W_form, W_vice):
    fi = form_idx.astype(jnp.int32).T.reshape(N)
    vi = vice_idx.astype(jnp.int32).T.reshape(N)
    out = _emb_gather(fi, vi, W_form, W_vice)
    return out.reshape(L, B, 2 * H).transpose(1, 0, 2)
